# baseline (speedup 1.0000x reference)
import functools

import jax
import jax.numpy as jnp
from jax import lax
from jax.experimental import pallas as pl
from jax.experimental.pallas import tpu as pltpu
from jax.experimental.pallas import tpu_sc as plsc

H = 512
W = 512
N = 20000
NC = 2
NS = 16
L = 16
NW = NC * NS
ROWS = H // NW
SLAB = ROWS * W
CHUNKS = N // L
CAP = 1024

_mesh = plsc.VectorSubcoreMesh(core_axis_name="c", subcore_axis_name="s")


@functools.partial(
    pl.kernel,
    out_type=(
        jax.ShapeDtypeStruct((H * W,), jnp.float32),
        jax.ShapeDtypeStruct((2 * H * W,), jnp.float32),
    ),
    mesh=_mesh,
    scratch_types=[
        pltpu.VMEM_SHARED((4 * N,), jnp.float32),
        pltpu.VMEM((4 * N,), jnp.float32),
        pltpu.VMEM((CAP,), jnp.int32),
        pltpu.VMEM((SLAB,), jnp.float32),
        pltpu.VMEM((2 * SLAB,), jnp.float32),
        pltpu.SemaphoreType.DMA,
    ],
    compiler_params=pltpu.CompilerParams(needs_layout_passes=False),
)
def _heatmap_sc(boxes_t_hbm, heat_hbm, size_hbm, bx_sh, bx, idxl, heat, size, sem):
    sid = lax.axis_index("s")
    wid = sid * NC + lax.axis_index("c")
    r0 = wid * ROWS
    base = r0 * W

    cp = pltpu.make_async_copy(boxes_t_hbm, bx_sh, sem)

    @pl.when(sid == 0)
    def _():
        cp.start()

    z = jnp.zeros((L,), jnp.float32)
    zi = jnp.zeros((L,), jnp.int32)

    def zero_body(j, carry):
        c = j * L
        heat[pl.ds(c, L)] = z
        size[pl.ds(c, L)] = z
        size[pl.ds(SLAB + c, L)] = z
        return carry

    lax.fori_loop(0, SLAB // L, zero_body, 0, unroll=8)

    def zero_idx_body(j, carry):
        idxl[pl.ds(j * L, L)] = zi
        return carry

    lax.fori_loop(0, CAP // L, zero_idx_body, 0, unroll=8)

    @pl.when(sid == 0)
    def _():
        cp.wait()

    plsc.subcore_barrier()
    pltpu.sync_copy(bx_sh, bx)

    iota = lax.broadcasted_iota(jnp.int32, (L,), 0)

    def scan_body(i, pos):
        b = i * L
        ys = bx[pl.ds(N + b, L)]
        cy = (ys * H).astype(jnp.int32)
        t = cy - r0
        m = t.astype(jnp.uint32) < ROWS
        mi = m.astype(jnp.int32)
        posv = pos + plsc.cumsum(mi) - mi
        ms = m & (posv.astype(jnp.uint32) < CAP)
        plsc.store_scatter(idxl, [posv], iota + b, mask=ms)
        return pos + plsc.all_reduce_population_count(m)

    pos = lax.fori_loop(0, CHUNKS, scan_body, jnp.zeros((L,), jnp.int32),
                        unroll=2)

    ones = jnp.ones((L,), jnp.float32)

    def emit_body(i, carry):
        jl = idxl[pl.ds(i * L, L)]
        xs = plsc.load_gather(bx, [jl])
        ys = plsc.load_gather(bx, [jl + N])
        ws = plsc.load_gather(bx, [jl + 2 * N])
        hs = plsc.load_gather(bx, [jl + 3 * N])
        m = (i * L + iota) < pos
        cx = (xs * W).astype(jnp.int32)
        cy = (ys * H).astype(jnp.int32)
        off = cy * W + cx - base
        plsc.store_scatter(heat, [off], ones, mask=m)
        plsc.store_scatter(size, [off], ws, mask=m)
        plsc.store_scatter(size, [off + SLAB], hs, mask=m)
        return carry

    lax.fori_loop(0, CAP // L, emit_body, 0, unroll=2)

    pltpu.sync_copy(heat, heat_hbm.at[pl.ds(base, SLAB)])
    pltpu.sync_copy(size.at[pl.ds(0, SLAB)], size_hbm.at[pl.ds(base, SLAB)])
    pltpu.sync_copy(
        size.at[pl.ds(SLAB, SLAB)], size_hbm.at[pl.ds(H * W + base, SLAB)]
    )


def kernel(boxes):
    boxes_t = boxes.T.reshape(-1)
    heat, size = _heatmap_sc(boxes_t)
    return heat.reshape(1, 1, H, W), size.reshape(1, 2, H, W)
